# Initial kernel scaffold; baseline (speedup 1.0000x reference)
import functools

import jax
import jax.numpy as jnp
from jax import lax
from jax.experimental import pallas as pl
from jax.experimental.pallas import tpu as pltpu
from jax.experimental.pallas import tpu_sc as plsc

F = 40960
H = 256
B = 16384
L = 32
NO = 32

NC, NS = 2, 16
NW = NC * NS
S = B // NW
C = 8
NCHUNK = S // C


def _proj_body(tb_ref, wa_ref, wb_ref, pa_ref, pb_ref):
    tb = tb_ref[...]
    pa_ref[...] = jnp.dot(tb, wa_ref[...], preferred_element_type=jnp.float32)
    pb_ref[...] = jnp.dot(tb, wb_ref[...], preferred_element_type=jnp.float32)


def _project(table, w1at, w1bt):
    bm = 2048
    grid = (F // bm,)
    return pl.pallas_call(
        _proj_body,
        grid=grid,
        in_specs=[
            pl.BlockSpec((bm, H), lambda m: (m, 0)),
            pl.BlockSpec((H, NO), lambda m: (0, 0)),
            pl.BlockSpec((H, NO), lambda m: (0, 0)),
        ],
        out_specs=[
            pl.BlockSpec((bm, NO), lambda m: (m, 0)),
            pl.BlockSpec((bm, NO), lambda m: (m, 0)),
        ],
        out_shape=[
            jax.ShapeDtypeStruct((F, NO), jnp.float32),
            jax.ShapeDtypeStruct((F, NO), jnp.float32),
        ],
    )(table, w1at, w1bt)


def _bag_body(pa_hbm, pb_hbm, us_hbm, them_hbm, out_hbm,
              idxu, idxt, bu0, bu1, bt0, bt1, acc,
              su0, su1, st0, st1):
    wid = lax.axis_index("s") * NC + lax.axis_index("c")
    base = wid * S

    pltpu.sync_copy(us_hbm.at[pl.ds(base, S)], idxu)
    pltpu.sync_copy(them_hbm.at[pl.ds(base, S)], idxt)

    bufs = ((bu0, bt0, su0, st0), (bu1, bt1, su1, st1))

    def start(c, slot):
        bu, bt, su, st = bufs[slot]
        pltpu.async_copy(pa_hbm.at[idxu.at[pl.ds(c * C, C)]], bu, su)
        pltpu.async_copy(pb_hbm.at[idxt.at[pl.ds(c * C, C)]], bt, st)

    def wait(slot):
        bu, bt, su, st = bufs[slot]
        pltpu.make_async_copy(pa_hbm.at[idxu.at[pl.ds(0, C)]], bu, su).wait()
        pltpu.make_async_copy(pb_hbm.at[idxt.at[pl.ds(0, C)]], bt, st).wait()

    def compute(c, slot):
        bu, bt, _, _ = bufs[slot]

        def sample(s, _):
            a0 = bu[s, 0, 0:16]
            a1 = bu[s, 0, 16:32]
            for r in range(1, L):
                a0 = a0 + bu[s, r, 0:16]
                a1 = a1 + bu[s, r, 16:32]
            for r in range(L):
                a0 = a0 + bt[s, r, 0:16]
                a1 = a1 + bt[s, r, 16:32]
            g = c * C + s
            acc[g, 0:16] = a0
            acc[g, 16:32] = a1
            return 0

        lax.fori_loop(0, C, sample, 0, unroll=2)

    start(0, 0)

    def pair(i, _):
        c0 = 2 * i
        start(c0 + 1, 1)
        wait(0)
        compute(c0, 0)
        start(jnp.minimum(c0 + 2, NCHUNK - 1), 0)
        wait(1)
        compute(c0 + 1, 1)
        return 0

    lax.fori_loop(0, NCHUNK // 2, pair, 0)
    wait(0)

    pltpu.sync_copy(acc, out_hbm.at[pl.ds(base, S)])


@functools.partial(
    pl.kernel,
    out_type=jax.ShapeDtypeStruct((B, NO), jnp.float32),
    mesh=plsc.VectorSubcoreMesh(core_axis_name="c", subcore_axis_name="s",
                                num_cores=NC, num_subcores=NS),
    scratch_types=[
        pltpu.VMEM((S, L), jnp.int32),
        pltpu.VMEM((S, L), jnp.int32),
        pltpu.VMEM((C, L, NO), jnp.float32),
        pltpu.VMEM((C, L, NO), jnp.float32),
        pltpu.VMEM((C, L, NO), jnp.float32),
        pltpu.VMEM((C, L, NO), jnp.float32),
        pltpu.VMEM((S, NO), jnp.float32),
        pltpu.SemaphoreType.DMA,
        pltpu.SemaphoreType.DMA,
        pltpu.SemaphoreType.DMA,
        pltpu.SemaphoreType.DMA,
    ],
)
def _bag(pa, pb, us, them, out, *rest):
    _bag_body(pa, pb, us, them, out, *rest)


def _head_body(x_ref, b1_ref, w2_ref, b2_ref, w3_ref, b3_ref, o_ref):
    x = jnp.clip(x_ref[...] + b1_ref[...], 0.0, 1.0)
    x = jnp.clip(
        jnp.dot(x, w2_ref[...].T, preferred_element_type=jnp.float32)
        + b2_ref[...], 0.0, 1.0)
    y = jnp.dot(x, w3_ref[...].T, preferred_element_type=jnp.float32)
    o_ref[...] = jnp.tanh(y + b3_ref[...])


def _head(x, b1, w2, b2, w3, b3):
    bm = 2048
    grid = (B // bm,)
    return pl.pallas_call(
        _head_body,
        grid=grid,
        in_specs=[
            pl.BlockSpec((bm, NO), lambda m: (m, 0)),
            pl.BlockSpec((NO,), lambda m: (0,)),
            pl.BlockSpec((NO, NO), lambda m: (0, 0)),
            pl.BlockSpec((NO,), lambda m: (0,)),
            pl.BlockSpec((1, NO), lambda m: (0, 0)),
            pl.BlockSpec((1,), lambda m: (0,)),
        ],
        out_specs=pl.BlockSpec((bm, 1), lambda m: (m, 0)),
        out_shape=jax.ShapeDtypeStruct((B, 1), jnp.float32),
    )(x, b1, w2, b2, w3, b3)


def kernel(us, them, table, W1, b1, W2, b2, W3, b3):
    w1at = W1[:, :H].T
    w1bt = W1[:, H:].T
    pa, pb = _project(table[:F], w1at, w1bt)
    preact = _bag(pa, pb, us, them)
    return _head(preact, b1, W2, b2, W3, b3)


# speedup vs baseline: 19.2599x; 19.2599x over previous
import functools

import jax
import jax.numpy as jnp
from jax import lax
from jax.experimental import pallas as pl
from jax.experimental.pallas import tpu as pltpu
from jax.experimental.pallas import tpu_sc as plsc

F = 40960
H = 256
B = 16384
L = 32
NO = 32

NC, NS = 2, 16
NW = NC * NS
S = B // NW
C = 8
NCHUNK = S // C


def _proj_body(tb_ref, wa_ref, wb_ref, pa_ref, pb_ref):
    tb = tb_ref[...]
    pa_ref[...] = jnp.dot(tb, wa_ref[...], preferred_element_type=jnp.float32)
    pb_ref[...] = jnp.dot(tb, wb_ref[...], preferred_element_type=jnp.float32)


def _project(table, w1at, w1bt):
    bm = 2048
    grid = (F // bm,)
    return pl.pallas_call(
        _proj_body,
        grid=grid,
        in_specs=[
            pl.BlockSpec((bm, H), lambda m: (m, 0)),
            pl.BlockSpec((H, NO), lambda m: (0, 0)),
            pl.BlockSpec((H, NO), lambda m: (0, 0)),
        ],
        out_specs=[
            pl.BlockSpec((bm, NO), lambda m: (m, 0)),
            pl.BlockSpec((bm, NO), lambda m: (m, 0)),
        ],
        out_shape=[
            jax.ShapeDtypeStruct((F, NO), jnp.float32),
            jax.ShapeDtypeStruct((F, NO), jnp.float32),
        ],
    )(table, w1at, w1bt)


def _bag_body(pa_hbm, pb_hbm, us_hbm, them_hbm, out_hbm,
              idxu, idxt, bu0, bu1, bt0, bt1, acc,
              su0, su1, st0, st1):
    wid = lax.axis_index("s") * NC + lax.axis_index("c")
    base = wid * S

    pltpu.sync_copy(us_hbm.at[pl.ds(base * L, S * L)], idxu)
    pltpu.sync_copy(them_hbm.at[pl.ds(base * L, S * L)], idxt)

    bufs = ((bu0, bt0, su0, st0), (bu1, bt1, su1, st1))

    def start(c, slot):
        bu, bt, su, st = bufs[slot]
        pltpu.async_copy(pa_hbm.at[idxu.at[pl.ds(c * C * L, C * L)]], bu, su)
        pltpu.async_copy(pb_hbm.at[idxt.at[pl.ds(c * C * L, C * L)]], bt, st)

    def wait(slot):
        bu, bt, su, st = bufs[slot]
        pltpu.make_async_copy(pa_hbm.at[idxu.at[pl.ds(0, C * L)]], bu, su).wait()
        pltpu.make_async_copy(pb_hbm.at[idxt.at[pl.ds(0, C * L)]], bt, st).wait()

    def compute(c, slot):
        bu, bt, _, _ = bufs[slot]

        def sample(s, _):
            a0 = bu[s * L, 0:16]
            a1 = bu[s * L, 16:32]
            for r in range(1, L):
                a0 = a0 + bu[s * L + r, 0:16]
                a1 = a1 + bu[s * L + r, 16:32]
            for r in range(L):
                a0 = a0 + bt[s * L + r, 0:16]
                a1 = a1 + bt[s * L + r, 16:32]
            g = c * C + s
            acc[g, 0:16] = a0
            acc[g, 16:32] = a1
            return 0

        lax.fori_loop(0, C, sample, 0, unroll=2)

    start(0, 0)

    def pair(i, _):
        c0 = 2 * i
        start(c0 + 1, 1)
        wait(0)
        compute(c0, 0)
        start(jnp.minimum(c0 + 2, NCHUNK - 1), 0)
        wait(1)
        compute(c0 + 1, 1)
        return 0

    lax.fori_loop(0, NCHUNK // 2, pair, 0)
    wait(0)

    pltpu.sync_copy(acc, out_hbm.at[pl.ds(base, S)])


@functools.cache
def _make_bag():
    @functools.partial(
        pl.kernel,
        out_type=jax.ShapeDtypeStruct((B, NO), jnp.float32),
        mesh=plsc.VectorSubcoreMesh(core_axis_name="c", subcore_axis_name="s",
                                    num_cores=NC, num_subcores=NS),
        scratch_types=[
            pltpu.VMEM((S * L,), jnp.int32),
            pltpu.VMEM((S * L,), jnp.int32),
            pltpu.VMEM((C * L, NO), jnp.float32),
            pltpu.VMEM((C * L, NO), jnp.float32),
            pltpu.VMEM((C * L, NO), jnp.float32),
            pltpu.VMEM((C * L, NO), jnp.float32),
            pltpu.VMEM((S, NO), jnp.float32),
            pltpu.SemaphoreType.DMA,
            pltpu.SemaphoreType.DMA,
            pltpu.SemaphoreType.DMA,
            pltpu.SemaphoreType.DMA,
        ],
        compiler_params=pltpu.CompilerParams(use_tc_tiling_on_sc=False),
    )
    def _bag(pa, pb, us, them, out, *rest):
        _bag_body(pa, pb, us, them, out, *rest)

    return _bag


def _head_body(x_ref, b1_ref, w2_ref, b2_ref, w3_ref, b3_ref, o_ref):
    x = jnp.clip(x_ref[...] + b1_ref[...], 0.0, 1.0)
    x = jnp.clip(
        jnp.dot(x, w2_ref[...].T, preferred_element_type=jnp.float32)
        + b2_ref[...], 0.0, 1.0)
    y = jnp.sum(x * w3_ref[...], axis=1, keepdims=True)
    o_ref[...] = jnp.tanh(y + b3_ref[0])


def _head(x, b1, w2, b2, w3, b3):
    bm = 2048
    grid = (B // bm,)
    return pl.pallas_call(
        _head_body,
        grid=grid,
        in_specs=[
            pl.BlockSpec((bm, NO), lambda m: (m, 0)),
            pl.BlockSpec((NO,), lambda m: (0,)),
            pl.BlockSpec((NO, NO), lambda m: (0, 0)),
            pl.BlockSpec((NO,), lambda m: (0,)),
            pl.BlockSpec((1, NO), lambda m: (0, 0)),
            pl.BlockSpec((1,), lambda m: (0,)),
        ],
        out_specs=pl.BlockSpec((bm, 1), lambda m: (m, 0)),
        out_shape=jax.ShapeDtypeStruct((B, 1), jnp.float32),
    )(x, b1, w2, b2, w3, b3)


def kernel(us, them, table, W1, b1, W2, b2, W3, b3):
    w1at = W1[:, :H].T
    w1bt = W1[:, H:].T
    pa, pb = _project(table[:F], w1at, w1bt)
    preact = _make_bag()(pa, pb, us.reshape(-1), them.reshape(-1))
    return _head(preact, b1, W2, b2, W3, b3)


# speedup vs baseline: 22.2014x; 1.1527x over previous
import functools

import jax
import jax.numpy as jnp
from jax import lax
from jax.experimental import pallas as pl
from jax.experimental.pallas import tpu as pltpu
from jax.experimental.pallas import tpu_sc as plsc

F = 40960
H = 256
B = 16384
L = 32
NO = 32

NC, NS = 2, 16
NW = NC * NS
S = B // NW
C = 8
NCHUNK = S // C


def _proj_body(tb_ref, w1_ref, pa_ref, pb_ref):
    tb = tb_ref[...]
    w1 = w1_ref[...]
    cdims = (((1,), (1,)), ((), ()))
    pa_ref[...] = lax.dot_general(tb, w1[:, :H], cdims,
                                  preferred_element_type=jnp.float32)
    pb_ref[...] = lax.dot_general(tb, w1[:, H:], cdims,
                                  preferred_element_type=jnp.float32)


def _project(table, w1):
    bm = 2048
    grid = (F // bm,)
    return pl.pallas_call(
        _proj_body,
        grid=grid,
        in_specs=[
            pl.BlockSpec((bm, H), lambda m: (m, 0)),
            pl.BlockSpec((NO, 2 * H), lambda m: (0, 0)),
        ],
        out_specs=[
            pl.BlockSpec((bm, NO), lambda m: (m, 0)),
            pl.BlockSpec((bm, NO), lambda m: (m, 0)),
        ],
        out_shape=[
            jax.ShapeDtypeStruct((F, NO), jnp.float32),
            jax.ShapeDtypeStruct((F, NO), jnp.float32),
        ],
    )(table, w1)


def _bag_body(pa_hbm, pb_hbm, us_hbm, them_hbm, out_hbm,
              idxu, idxt, bu0, bu1, bt0, bt1, acc,
              su0, su1, st0, st1):
    wid = lax.axis_index("s") * NC + lax.axis_index("c")
    base = wid * S
    rows_per_w = S * L // 128

    pltpu.sync_copy(us_hbm.at[pl.ds(wid * rows_per_w, rows_per_w)], idxu)
    pltpu.sync_copy(them_hbm.at[pl.ds(wid * rows_per_w, rows_per_w)], idxt)

    bufs = ((bu0, bt0, su0, st0), (bu1, bt1, su1, st1))
    RPC = C * L // 128

    def start(c, slot):
        bu, bt, su, st = bufs[slot]
        for j in range(RPC):
            dst = pl.ds(j * 128, 128)
            pltpu.async_copy(pa_hbm.at[idxu.at[c * RPC + j]], bu.at[dst], su)
            pltpu.async_copy(pb_hbm.at[idxt.at[c * RPC + j]], bt.at[dst], st)

    def wait(slot):
        bu, bt, su, st = bufs[slot]
        for j in range(RPC):
            dst = pl.ds(j * 128, 128)
            pltpu.make_async_copy(pa_hbm.at[idxu.at[0]], bu.at[dst], su).wait()
            pltpu.make_async_copy(pb_hbm.at[idxt.at[0]], bt.at[dst], st).wait()

    def compute(c, slot):
        bu, bt, _, _ = bufs[slot]

        def sample(s, _):
            a0 = bu[s * L, 0:16]
            a1 = bu[s * L, 16:32]
            for r in range(1, L):
                a0 = a0 + bu[s * L + r, 0:16]
                a1 = a1 + bu[s * L + r, 16:32]
            for r in range(L):
                a0 = a0 + bt[s * L + r, 0:16]
                a1 = a1 + bt[s * L + r, 16:32]
            g = c * C + s
            acc[g, 0:16] = a0
            acc[g, 16:32] = a1
            return 0

        lax.fori_loop(0, C, sample, 0, unroll=2)

    start(0, 0)

    def pair(i, _):
        c0 = 2 * i
        start(c0 + 1, 1)
        wait(0)
        compute(c0, 0)
        start(jnp.minimum(c0 + 2, NCHUNK - 1), 0)
        wait(1)
        compute(c0 + 1, 1)
        return 0

    lax.fori_loop(0, NCHUNK // 2, pair, 0)
    wait(0)

    pltpu.sync_copy(acc, out_hbm.at[pl.ds(base, S)])


@functools.cache
def _make_bag():
    @functools.partial(
        pl.kernel,
        out_type=jax.ShapeDtypeStruct((B, NO), jnp.float32),
        mesh=plsc.VectorSubcoreMesh(core_axis_name="c", subcore_axis_name="s",
                                    num_cores=NC, num_subcores=NS),
        scratch_types=[
            pltpu.VMEM((S * L // 128, 128), jnp.int32),
            pltpu.VMEM((S * L // 128, 128), jnp.int32),
            pltpu.VMEM((C * L, NO), jnp.float32),
            pltpu.VMEM((C * L, NO), jnp.float32),
            pltpu.VMEM((C * L, NO), jnp.float32),
            pltpu.VMEM((C * L, NO), jnp.float32),
            pltpu.VMEM((S, NO), jnp.float32),
            pltpu.SemaphoreType.DMA,
            pltpu.SemaphoreType.DMA,
            pltpu.SemaphoreType.DMA,
            pltpu.SemaphoreType.DMA,
        ],
        compiler_params=pltpu.CompilerParams(use_tc_tiling_on_sc=False),
    )
    def _bag(pa, pb, us, them, out, *rest):
        _bag_body(pa, pb, us, them, out, *rest)

    return _bag


def _head_body(x_ref, b1_ref, w2_ref, b2_ref, w3_ref, b3_ref, o_ref):
    x = jnp.clip(x_ref[...] + b1_ref[...], 0.0, 1.0)
    x = jnp.clip(
        jnp.dot(x, w2_ref[...].T, preferred_element_type=jnp.float32)
        + b2_ref[...], 0.0, 1.0)
    y = jnp.sum(x * w3_ref[...], axis=1, keepdims=True)
    o_ref[...] = jnp.tanh(y + b3_ref[0])


def _head(x, b1, w2, b2, w3, b3):
    bm = 2048
    grid = (B // bm,)
    return pl.pallas_call(
        _head_body,
        grid=grid,
        in_specs=[
            pl.BlockSpec((bm, NO), lambda m: (m, 0)),
            pl.BlockSpec((NO,), lambda m: (0,)),
            pl.BlockSpec((NO, NO), lambda m: (0, 0)),
            pl.BlockSpec((NO,), lambda m: (0,)),
            pl.BlockSpec((1, NO), lambda m: (0, 0)),
            pl.BlockSpec((1,), lambda m: (0,)),
        ],
        out_specs=pl.BlockSpec((bm, 1), lambda m: (m, 0)),
        out_shape=jax.ShapeDtypeStruct((B, 1), jnp.float32),
    )(x, b1, w2, b2, w3, b3)


def kernel(us, them, table, W1, b1, W2, b2, W3, b3):
    pa, pb = _project(table, W1)
    us_lin = us.reshape(B * L // 128, 128)
    them_lin = them.reshape(B * L // 128, 128)
    preact = _make_bag()(pa, pb, us_lin, them_lin)
    return _head(preact, b1, W2, b2, W3, b3)


# speedup vs baseline: 35.0053x; 1.5767x over previous
import functools

import jax
import jax.numpy as jnp
from jax import lax
from jax.experimental import pallas as pl
from jax.experimental.pallas import tpu as pltpu
from jax.experimental.pallas import tpu_sc as plsc

F = 40960
H = 256
B = 16384
L = 32
NO = 32

NC, NS = 2, 16
NW = NC * NS
S = B // NW
C = 16
NCHUNK = S // C


EB = 256


def _pack_bf16_words(r32):
    w = lax.bitcast_convert_type(r32, jnp.int32)
    rnd = w + jnp.int32(0x7FFF) + ((w >> 16) & jnp.int32(1))
    lo = (rnd[:, 0:16] >> 16) & jnp.int32(0xFFFF)
    hi = rnd[:, 16:32] & jnp.int32(-65536)
    return lo | hi


def _proj_body(tb_ref, w1_ref, pa_ref, pb_ref):
    w1 = w1_ref[...]
    cdims = (((1,), (1,)), ((), ()))
    pas, pbs = [], []
    for g in range(16):
        q = tb_ref[pl.ds(g * EB, EB), :]
        pas.append(_pack_bf16_words(
            lax.dot_general(q, w1[:, :H], cdims,
                            preferred_element_type=jnp.float32)))
        pbs.append(_pack_bf16_words(
            lax.dot_general(q, w1[:, H:], cdims,
                            preferred_element_type=jnp.float32)))
    pa_ref[...] = jnp.concatenate(
        [jnp.concatenate(pas[:8], axis=1), jnp.concatenate(pas[8:], axis=1)],
        axis=0)
    pb_ref[...] = jnp.concatenate(
        [jnp.concatenate(pbs[:8], axis=1), jnp.concatenate(pbs[8:], axis=1)],
        axis=0)


def _project(table, w1):
    bm = 16 * EB
    grid = (F // bm,)
    return pl.pallas_call(
        _proj_body,
        grid=grid,
        in_specs=[
            pl.BlockSpec((bm, H), lambda m: (m, 0)),
            pl.BlockSpec((NO, 2 * H), lambda m: (0, 0)),
        ],
        out_specs=[
            pl.BlockSpec((2 * EB, 128), lambda m: (m, 0)),
            pl.BlockSpec((2 * EB, 128), lambda m: (m, 0)),
        ],
        out_shape=[
            jax.ShapeDtypeStruct((F // 8, 128), jnp.int32),
            jax.ShapeDtypeStruct((F // 8, 128), jnp.int32),
        ],
    )(table, w1)


NSLOT = 4


def _bag_body(pa_hbm, pb_hbm, us_hbm, them_hbm, out_hbm,
              idxu, idxt, bu0, bu1, bu2, bu3, bt0, bt1, bt2, bt3, acc,
              s0, s1, s2, s3):
    wid = lax.axis_index("s") * NC + lax.axis_index("c")
    base = wid * S
    rows_per_w = S * L // 128

    pltpu.sync_copy(us_hbm.at[pl.ds(wid * rows_per_w, rows_per_w)], idxu)
    pltpu.sync_copy(them_hbm.at[pl.ds(wid * rows_per_w, rows_per_w)], idxt)

    bufs = ((bu0, bt0, s0), (bu1, bt1, s1), (bu2, bt2, s2), (bu3, bt3, s3))
    RPC = C * L // 128

    def start(c, slot):
        bu, bt, sem = bufs[slot]
        for j in range(RPC):
            dst = pl.ds(j * 128, 128)
            pltpu.async_copy(pa_hbm.at[idxu.at[c * RPC + j]], bu.at[dst], sem)
            pltpu.async_copy(pb_hbm.at[idxt.at[c * RPC + j]], bt.at[dst], sem)

    def wait(slot):
        bu, bt, sem = bufs[slot]
        for j in range(RPC):
            dst = pl.ds(j * 128, 128)
            pltpu.make_async_copy(pa_hbm.at[idxu.at[0]], bu.at[dst], sem).wait()
            pltpu.make_async_copy(pb_hbm.at[idxt.at[0]], bt.at[dst], sem).wait()

    def compute(c, slot):
        bu, bt, _ = bufs[slot]
        mask = jnp.int32(-65536)

        def row(buf, s, r):
            w = buf[s * L + r, 0:16]
            lo = plsc.bitcast(w << 16, jnp.float32)
            hi = plsc.bitcast(w & mask, jnp.float32)
            return lo, hi

        def sample(s, _):
            a0, a1 = row(bu, s, 0)
            b0, b1 = row(bt, s, 0)
            for r in range(1, L):
                lo, hi = row(bu, s, r)
                a0 = a0 + lo
                a1 = a1 + hi
                lo, hi = row(bt, s, r)
                b0 = b0 + lo
                b1 = b1 + hi
            g = c * C + s
            acc[g, 0:16] = a0 + b0
            acc[g, 16:32] = a1 + b1
            return 0

        lax.fori_loop(0, C, sample, 0, unroll=2)

    for c in range(NSLOT - 1):
        start(c, c)

    def quad(i, _):
        for b in range(NSLOT):
            c = NSLOT * i + b
            wait(b)
            compute(c, b)
            start(jnp.minimum(c + NSLOT - 1, NCHUNK - 1), (b + NSLOT - 1) % NSLOT)
        return 0

    lax.fori_loop(0, NCHUNK // NSLOT, quad, 0)
    for b in range(NSLOT - 1):
        wait(b)

    pltpu.sync_copy(acc, out_hbm.at[pl.ds(base, S)])


@functools.cache
def _make_bag():
    @functools.partial(
        pl.kernel,
        out_type=jax.ShapeDtypeStruct((B, NO), jnp.float32),
        mesh=plsc.VectorSubcoreMesh(core_axis_name="c", subcore_axis_name="s",
                                    num_cores=NC, num_subcores=NS),
        scratch_types=[
            pltpu.VMEM((S * L // 128, 128), jnp.int32),
            pltpu.VMEM((S * L // 128, 128), jnp.int32),
            pltpu.VMEM((C * L, 16), jnp.int32),
            pltpu.VMEM((C * L, 16), jnp.int32),
            pltpu.VMEM((C * L, 16), jnp.int32),
            pltpu.VMEM((C * L, 16), jnp.int32),
            pltpu.VMEM((C * L, 16), jnp.int32),
            pltpu.VMEM((C * L, 16), jnp.int32),
            pltpu.VMEM((C * L, 16), jnp.int32),
            pltpu.VMEM((C * L, 16), jnp.int32),
            pltpu.VMEM((S, NO), jnp.float32),
            pltpu.SemaphoreType.DMA,
            pltpu.SemaphoreType.DMA,
            pltpu.SemaphoreType.DMA,
            pltpu.SemaphoreType.DMA,
        ],
        compiler_params=pltpu.CompilerParams(use_tc_tiling_on_sc=False,
                                             needs_layout_passes=False),
    )
    def _bag(pa, pb, us, them, out, *rest):
        _bag_body(pa, pb, us, them, out, *rest)

    return _bag


def _head_body(x_ref, b1_ref, w2k_ref, b2_ref, w3k_ref, b3_ref, o_ref):
    x = jnp.clip(x_ref[...] + b1_ref[...], 0.0, 1.0)
    x = jnp.clip(
        jnp.dot(x, w2k_ref[...], preferred_element_type=jnp.float32)
        + b2_ref[...], 0.0, 1.0)
    y = jnp.dot(x, w3k_ref[...], preferred_element_type=jnp.float32)
    o_ref[...] = jnp.tanh(y + b3_ref[0])


def _head(xp, b1, w2, b2, w3, b3):
    eye4 = jnp.eye(4, dtype=jnp.float32)
    w2k = jnp.kron(eye4, w2.T)
    w3k = jnp.kron(eye4, w3.T)
    b1t = jnp.tile(b1, 4)
    b2t = jnp.tile(b2, 4)
    bm = 512
    grid = (B // 4 // bm,)
    yp = pl.pallas_call(
        _head_body,
        grid=grid,
        in_specs=[
            pl.BlockSpec((bm, 128), lambda m: (m, 0)),
            pl.BlockSpec((128,), lambda m: (0,)),
            pl.BlockSpec((128, 128), lambda m: (0, 0)),
            pl.BlockSpec((128,), lambda m: (0,)),
            pl.BlockSpec((128, 4), lambda m: (0, 0)),
            pl.BlockSpec((1,), lambda m: (0,)),
        ],
        out_specs=pl.BlockSpec((bm, 4), lambda m: (m, 0)),
        out_shape=jax.ShapeDtypeStruct((B // 4, 4), jnp.float32),
    )(xp, b1t, w2k, b2t, w3k, b3)
    return yp.reshape(B, 1)


def _remap(idx):
    return ((idx & -2048) + ((idx & 255) << 3) + ((idx >> 8) & 7)).reshape(
        B * L // 128, 128)


def kernel(us, them, table, W1, b1, W2, b2, W3, b3):
    pa8, pb8 = _project(table, W1)
    pa = pa8.reshape(F, 16)
    pb = pb8.reshape(F, 16)
    us_lin = _remap(us)
    them_lin = _remap(them)
    preact = _make_bag()(pa, pb, us_lin, them_lin)
    return _head(preact.reshape(B // 4, 128), b1, W2, b2, W3, b3)
